# baseline (speedup 1.0000x reference)
import functools

import jax
import jax.numpy as jnp
from jax import lax
from jax.experimental import pallas as pl
from jax.experimental.pallas import tpu as pltpu
from jax.experimental.pallas import tpu_sc as plsc

NUM_CORES = 2
NUM_SUBCORES = 16
NUM_WORKERS = NUM_CORES * NUM_SUBCORES
LANES = 16

BS = 4
NA = 256
NB = 256
DIM = 128
PAIRS = BS * NA
PAIRS_PER_W = PAIRS // NUM_WORKERS
VOCAB = 10
NBUF = 2
GROWS = 144
VROWS = NB - GROWS


def _bucketize(d):
    one = jnp.full((LANES,), 1, jnp.int32)
    five = jnp.full((LANES,), 5, jnp.int32)
    b = jnp.minimum(d, five)
    for sh in (3, 4, 5, 6):
        b = b + jnp.minimum(lax.shift_right_logical(d, sh), one)
    return b


def _body(sa0_hbm, sa1_hbm, sb0_hbm, sb1_hbm, w_hbm, out_hbm,
          sa0_v, sa1_v, sb0_v, sb1_v, w_sh, w_v, idx_v, rows_v,
          gsem0, gsem1, osem0, osem1):
    gsem = (gsem0, gsem1)
    osem = (osem0, osem1)
    wid = lax.axis_index("c") * NUM_SUBCORES + lax.axis_index("s")
    pair_base = wid * PAIRS_PER_W
    bsi = pair_base // NA

    pltpu.sync_copy(sa0_hbm.at[pl.ds(pair_base, PAIRS_PER_W)],
                    sa0_v.at[pl.ds(0, PAIRS_PER_W)])
    pltpu.sync_copy(sa1_hbm.at[pl.ds(pair_base, PAIRS_PER_W)],
                    sa1_v.at[pl.ds(0, PAIRS_PER_W)])
    pltpu.sync_copy(sb0_hbm.at[bsi], sb0_v)
    pltpu.sync_copy(sb1_hbm.at[bsi], sb1_v)
    pltpu.sync_copy(w_hbm, w_v)

    @pl.when(lax.axis_index("s") == 0)
    def _():
        pltpu.sync_copy(w_hbm, w_sh)

    plsc.subcore_barrier()

    def splat_span(ref, j):
        return jnp.full((LANES,), ref[pl.ds(j, LANES)][0], jnp.int32)

    def buckets_for(a0, a1, g):
        sb0 = sb0_v[pl.ds(g * LANES, LANES)]
        sb1 = sb1_v[pl.ds(g * LANES, LANES)]
        d = jnp.minimum(jnp.abs(sb0 - a1), jnp.abs(a0 - sb1))
        return _bucketize(d)

    def out_issue(p, buf):
        pltpu.async_copy(
            rows_v.at[buf], out_hbm.at[pl.ds(p * NB, NB)], osem[buf])

    def out_wait(buf):
        pltpu.make_async_copy(
            rows_v.at[buf], out_hbm.at[pl.ds(0, NB)], osem[buf]).wait()

    def two_pairs(jo, carry):
        for buf in range(NBUF):
            j = NBUF * jo + buf
            a0 = splat_span(sa0_v, j)
            a1 = splat_span(sa1_v, j)
            for g in range(GROWS // LANES):
                idx_v[buf, pl.ds(g * LANES, LANES)] = buckets_for(a0, a1, g)
            @pl.when(jo > 0)
            def _():
                out_wait(buf)
            for lo in range(0, GROWS, 128):
                n = min(128, GROWS - lo)
                pltpu.async_copy(
                    w_sh.at[idx_v.at[buf, pl.ds(lo, n)]],
                    rows_v.at[buf, pl.ds(lo, n)], gsem[buf])
            for g in range(VROWS // LANES):
                bv = buckets_for(a0, a1, (GROWS // LANES) + g)
                for lane in range(LANES):
                    b = bv[lane]
                    row = GROWS + g * LANES + lane
                    for c in range(DIM // LANES):
                        rows_v[buf, row, pl.ds(c * LANES, LANES)] = (
                            w_v[b, pl.ds(c * LANES, LANES)])
            for lo in range(0, GROWS, 128):
                n = min(128, GROWS - lo)
                pltpu.make_async_copy(
                    w_sh.at[idx_v.at[buf, pl.ds(lo, n)]],
                    rows_v.at[buf, pl.ds(lo, n)], gsem[buf]).wait()
            out_issue(pair_base + j, buf)
        return carry

    lax.fori_loop(0, PAIRS_PER_W // NBUF, two_pairs, 0)
    out_wait(0)
    out_wait(1)


@jax.jit
def kernel(spans_a, spans_b, W):
    sa0 = spans_a[..., 0].reshape(PAIRS)
    sa1 = spans_a[..., 1].reshape(PAIRS)
    sb0 = spans_b[..., 0]
    sb1 = spans_b[..., 1]

    mesh = plsc.VectorSubcoreMesh(core_axis_name="c", subcore_axis_name="s")
    run = functools.partial(
        pl.kernel,
        mesh=mesh,
        out_type=jax.ShapeDtypeStruct((PAIRS * NB, DIM), jnp.float32),
        scratch_types=[
            pltpu.VMEM((PAIRS_PER_W + LANES,), jnp.int32),
            pltpu.VMEM((PAIRS_PER_W + LANES,), jnp.int32),
            pltpu.VMEM((NB,), jnp.int32),
            pltpu.VMEM((NB,), jnp.int32),
            pltpu.VMEM_SHARED((VOCAB, DIM), jnp.float32),
            pltpu.VMEM((VOCAB, DIM), jnp.float32),
            pltpu.VMEM((NBUF, GROWS), jnp.int32),
            pltpu.VMEM((NBUF, NB, DIM), jnp.float32),
            pltpu.SemaphoreType.DMA,
            pltpu.SemaphoreType.DMA,
            pltpu.SemaphoreType.DMA,
            pltpu.SemaphoreType.DMA,
        ],
    )(_body)
    out = run(sa0, sa1, sb0, sb1, W)
    return out.reshape(BS, NA, NB, DIM)


# speedup vs baseline: 1.0551x; 1.0551x over previous
import functools

import jax
import jax.numpy as jnp
from jax import lax
from jax.experimental import pallas as pl
from jax.experimental.pallas import tpu as pltpu
from jax.experimental.pallas import tpu_sc as plsc

NUM_CORES = 2
NUM_SUBCORES = 16
NUM_WORKERS = NUM_CORES * NUM_SUBCORES
LANES = 16

BS = 4
NA = 256
NB = 256
DIM = 128
PAIRS = BS * NA
PAIRS_PER_W = PAIRS // NUM_WORKERS
VOCAB = 10
NBUF = 2
GROWS = 176
VROWS = NB - GROWS


def _bucketize(d):
    one = jnp.full((LANES,), 1, jnp.int32)
    five = jnp.full((LANES,), 5, jnp.int32)
    b = jnp.minimum(d, five)
    for sh in (3, 4, 5, 6):
        b = b + jnp.minimum(lax.shift_right_logical(d, sh), one)
    return b


def _body(sa0_hbm, sa1_hbm, sb0_hbm, sb1_hbm, w_hbm, out_hbm,
          sa0_v, sa1_v, sb0_v, sb1_v, w_sh, w_v, idx_v, rows_v,
          gsem0, gsem1, osem0, osem1):
    gsem = (gsem0, gsem1)
    osem = (osem0, osem1)
    wid = lax.axis_index("c") * NUM_SUBCORES + lax.axis_index("s")
    pair_base = wid * PAIRS_PER_W
    bsi = pair_base // NA

    pltpu.sync_copy(sa0_hbm.at[pl.ds(pair_base, PAIRS_PER_W)],
                    sa0_v.at[pl.ds(0, PAIRS_PER_W)])
    pltpu.sync_copy(sa1_hbm.at[pl.ds(pair_base, PAIRS_PER_W)],
                    sa1_v.at[pl.ds(0, PAIRS_PER_W)])
    pltpu.sync_copy(sb0_hbm.at[bsi], sb0_v)
    pltpu.sync_copy(sb1_hbm.at[bsi], sb1_v)
    pltpu.sync_copy(w_hbm, w_v)

    @pl.when(lax.axis_index("s") == 0)
    def _():
        pltpu.sync_copy(w_hbm, w_sh)

    plsc.subcore_barrier()

    def splat_span(ref, j):
        return jnp.full((LANES,), ref[pl.ds(j, LANES)][0], jnp.int32)

    def buckets_for(a0, a1, g):
        sb0 = sb0_v[pl.ds(g * LANES, LANES)]
        sb1 = sb1_v[pl.ds(g * LANES, LANES)]
        d = jnp.minimum(jnp.abs(sb0 - a1), jnp.abs(a0 - sb1))
        return _bucketize(d)

    def out_issue(p, buf):
        pltpu.async_copy(
            rows_v.at[buf], out_hbm.at[pl.ds(p * NB, NB)], osem[buf])

    def out_wait(buf):
        pltpu.make_async_copy(
            rows_v.at[buf], out_hbm.at[pl.ds(0, NB)], osem[buf]).wait()

    def gather_issue(buf):
        for lo in range(0, GROWS, 128):
            n = min(128, GROWS - lo)
            pltpu.async_copy(
                w_sh.at[idx_v.at[buf, pl.ds(lo, n)]],
                rows_v.at[buf, pl.ds(lo, n)], gsem[buf])

    def gather_wait(buf):
        for lo in range(0, GROWS, 128):
            n = min(128, GROWS - lo)
            pltpu.make_async_copy(
                w_sh.at[idx_v.at[buf, pl.ds(lo, n)]],
                rows_v.at[buf, pl.ds(lo, n)], gsem[buf]).wait()

    def two_pairs(jo, carry):
        for buf in range(NBUF):
            j = NBUF * jo + buf
            a0 = splat_span(sa0_v, j)
            a1 = splat_span(sa1_v, j)
            for g in range(GROWS // LANES):
                idx_v[buf, pl.ds(g * LANES, LANES)] = buckets_for(a0, a1, g)
            @pl.when(jo > 0)
            def _():
                out_wait(buf)
            gather_issue(buf)
            for g in range(VROWS // LANES):
                bv = buckets_for(a0, a1, (GROWS // LANES) + g)
                for lane in range(LANES):
                    b = bv[lane]
                    row = GROWS + g * LANES + lane
                    for c in range(DIM // LANES):
                        rows_v[buf, row, pl.ds(c * LANES, LANES)] = (
                            w_v[b, pl.ds(c * LANES, LANES)])
            other = (buf + NBUF - 1) % NBUF
            if buf == 0:
                @pl.when(jo > 0)
                def _():
                    gather_wait(other)
                    out_issue(pair_base + NBUF * jo - 1, other)
            else:
                gather_wait(other)
                out_issue(pair_base + NBUF * jo + buf - 1, other)
        return carry

    lax.fori_loop(0, PAIRS_PER_W // NBUF, two_pairs, 0)
    gather_wait(NBUF - 1)
    out_issue(pair_base + PAIRS_PER_W - 1, NBUF - 1)
    out_wait(0)
    out_wait(1)


@jax.jit
def kernel(spans_a, spans_b, W):
    sa0 = spans_a[..., 0].reshape(PAIRS)
    sa1 = spans_a[..., 1].reshape(PAIRS)
    sb0 = spans_b[..., 0]
    sb1 = spans_b[..., 1]

    mesh = plsc.VectorSubcoreMesh(core_axis_name="c", subcore_axis_name="s")
    run = functools.partial(
        pl.kernel,
        mesh=mesh,
        out_type=jax.ShapeDtypeStruct((PAIRS * NB, DIM), jnp.float32),
        scratch_types=[
            pltpu.VMEM((PAIRS_PER_W + LANES,), jnp.int32),
            pltpu.VMEM((PAIRS_PER_W + LANES,), jnp.int32),
            pltpu.VMEM((NB,), jnp.int32),
            pltpu.VMEM((NB,), jnp.int32),
            pltpu.VMEM_SHARED((VOCAB, DIM), jnp.float32),
            pltpu.VMEM((VOCAB, DIM), jnp.float32),
            pltpu.VMEM((NBUF, GROWS), jnp.int32),
            pltpu.VMEM((NBUF, NB, DIM), jnp.float32),
            pltpu.SemaphoreType.DMA,
            pltpu.SemaphoreType.DMA,
            pltpu.SemaphoreType.DMA,
            pltpu.SemaphoreType.DMA,
        ],
    )(_body)
    out = run(sa0, sa1, sb0, sb1, W)
    return out.reshape(BS, NA, NB, DIM)


# speedup vs baseline: 1.3452x; 1.2749x over previous
import functools

import jax
import jax.numpy as jnp
from jax import lax
from jax.experimental import pallas as pl
from jax.experimental.pallas import tpu as pltpu
from jax.experimental.pallas import tpu_sc as plsc

NUM_CORES = 2
NUM_SUBCORES = 16
NUM_WORKERS = NUM_CORES * NUM_SUBCORES
LANES = 16

BS = 4
NA = 256
NB = 256
DIM = 128
PAIRS = BS * NA
PAIRS_PER_W = PAIRS // NUM_WORKERS
VOCAB = 10
NBUF = 2
GROWS = 192
VROWS = NB - GROWS


def _bucketize(d):
    one = jnp.full((LANES,), 1, jnp.int32)
    five = jnp.full((LANES,), 5, jnp.int32)
    b = jnp.minimum(d, five)
    for sh in (3, 4, 5, 6):
        b = b + jnp.minimum(lax.shift_right_logical(d, sh), one)
    return b


def _body(sa0_hbm, sa1_hbm, sb0_hbm, sb1_hbm, w_hbm, out_hbm,
          sa0_v, sa1_v, sb0_v, sb1_v, w_sh, w_v, idx_v, rows_v,
          gsem0, gsem1, osem0, osem1):
    gsem = (gsem0, gsem1)
    osem = (osem0, osem1)
    wid = lax.axis_index("c") * NUM_SUBCORES + lax.axis_index("s")
    pair_base = wid * PAIRS_PER_W
    bsi = pair_base // NA

    pltpu.sync_copy(sa0_hbm.at[pl.ds(pair_base, PAIRS_PER_W)],
                    sa0_v.at[pl.ds(0, PAIRS_PER_W)])
    pltpu.sync_copy(sa1_hbm.at[pl.ds(pair_base, PAIRS_PER_W)],
                    sa1_v.at[pl.ds(0, PAIRS_PER_W)])
    pltpu.sync_copy(sb0_hbm.at[bsi], sb0_v)
    pltpu.sync_copy(sb1_hbm.at[bsi], sb1_v)
    pltpu.sync_copy(w_hbm, w_v)

    @pl.when(lax.axis_index("s") == 0)
    def _():
        pltpu.sync_copy(w_hbm, w_sh)

    plsc.subcore_barrier()

    def splat_span(ref, j):
        return jnp.full((LANES,), ref[pl.ds(j, LANES)][0], jnp.int32)

    def buckets_for(a0, a1, g):
        sb0 = sb0_v[pl.ds(g * LANES, LANES)]
        sb1 = sb1_v[pl.ds(g * LANES, LANES)]
        d = jnp.minimum(jnp.abs(sb0 - a1), jnp.abs(a0 - sb1))
        return _bucketize(d)

    def out_issue(p, buf):
        pltpu.async_copy(
            rows_v.at[buf], out_hbm.at[pl.ds(p * NB, NB)], osem[buf])

    def out_wait(buf):
        pltpu.make_async_copy(
            rows_v.at[buf], out_hbm.at[pl.ds(0, NB)], osem[buf]).wait()

    def two_pairs(jo, carry):
        for buf in range(NBUF):
            j = NBUF * jo + buf
            a0 = splat_span(sa0_v, j)
            a1 = splat_span(sa1_v, j)
            for g in range(GROWS // LANES):
                idx_v[buf, pl.ds(g * LANES, LANES)] = buckets_for(a0, a1, g)
            @pl.when(jo > 0)
            def _():
                out_wait(buf)
            for lo in range(0, GROWS, 128):
                n = min(128, GROWS - lo)
                pltpu.async_copy(
                    w_sh.at[idx_v.at[buf, pl.ds(lo, n)]],
                    rows_v.at[buf, pl.ds(lo, n)], gsem[buf])
            for g in range(VROWS // LANES):
                bv = buckets_for(a0, a1, (GROWS // LANES) + g)
                for lane in range(LANES):
                    b = bv[lane]
                    row = GROWS + g * LANES + lane
                    for c in range(DIM // LANES):
                        rows_v[buf, row, pl.ds(c * LANES, LANES)] = (
                            w_v[b, pl.ds(c * LANES, LANES)])
            for lo in range(0, GROWS, 128):
                n = min(128, GROWS - lo)
                pltpu.make_async_copy(
                    w_sh.at[idx_v.at[buf, pl.ds(lo, n)]],
                    rows_v.at[buf, pl.ds(lo, n)], gsem[buf]).wait()
            out_issue(pair_base + j, buf)
        return carry

    lax.fori_loop(0, PAIRS_PER_W // NBUF, two_pairs, 0)
    out_wait(0)
    out_wait(1)


@jax.jit
def kernel(spans_a, spans_b, W):
    sa0 = spans_a[..., 0].reshape(PAIRS)
    sa1 = spans_a[..., 1].reshape(PAIRS)
    sb0 = spans_b[..., 0]
    sb1 = spans_b[..., 1]

    mesh = plsc.VectorSubcoreMesh(core_axis_name="c", subcore_axis_name="s")
    run = functools.partial(
        pl.kernel,
        mesh=mesh,
        out_type=jax.ShapeDtypeStruct((PAIRS * NB, DIM), jnp.float32),
        scratch_types=[
            pltpu.VMEM((PAIRS_PER_W + LANES,), jnp.int32),
            pltpu.VMEM((PAIRS_PER_W + LANES,), jnp.int32),
            pltpu.VMEM((NB,), jnp.int32),
            pltpu.VMEM((NB,), jnp.int32),
            pltpu.VMEM_SHARED((VOCAB, DIM), jnp.float32),
            pltpu.VMEM((VOCAB, DIM), jnp.float32),
            pltpu.VMEM((NBUF, GROWS), jnp.int32),
            pltpu.VMEM((NBUF, NB, DIM), jnp.float32),
            pltpu.SemaphoreType.DMA,
            pltpu.SemaphoreType.DMA,
            pltpu.SemaphoreType.DMA,
            pltpu.SemaphoreType.DMA,
        ],
    )(_body)
    out = run(sa0, sa1, sb0, sb1, W)
    return out.reshape(BS, NA, NB, DIM)
